# baseline (device time: 149887 ns/iter reference)
import jax
import jax.numpy as jnp
from jax import lax
from jax.experimental import pallas as pl
from jax.experimental.pallas import tpu as pltpu


def kernel(x, assign, W1, W2):
    T, D = x.shape
    E, _, F = W1.shape
    assign2 = assign.reshape(T, 1)

    def peer_of():
        my_x = lax.axis_index("x")
        my_y = lax.axis_index("y")
        my_z = lax.axis_index("z")
        return my_y, (my_x, 1 - my_y, my_z)

    def peer_barrier(peer):
        barrier_sem = pltpu.get_barrier_semaphore()
        pl.semaphore_signal(
            barrier_sem,
            inc=1,
            device_id=peer,
            device_id_type=pl.DeviceIdType.MESH,
        )
        pl.semaphore_wait(barrier_sem, 1)

    def gather_body(x_ref, a_ref, xall_ref, aall_ref, send_sems, recv_sems):
        _, peer = peer_of()
        peer_barrier(peer)

        xall_ref[0] = x_ref[...]
        aall_ref[0] = a_ref[...]

        rdma_x = pltpu.make_async_remote_copy(
            src_ref=x_ref,
            dst_ref=xall_ref.at[1],
            send_sem=send_sems.at[0],
            recv_sem=recv_sems.at[0],
            device_id=peer,
            device_id_type=pl.DeviceIdType.MESH,
        )
        rdma_a = pltpu.make_async_remote_copy(
            src_ref=a_ref,
            dst_ref=aall_ref.at[1],
            send_sem=send_sems.at[1],
            recv_sem=recv_sems.at[1],
            device_id=peer,
            device_id_type=pl.DeviceIdType.MESH,
        )
        rdma_x.start()
        rdma_a.start()
        rdma_x.wait()
        rdma_a.wait()

    xall, aall = pl.pallas_call(
        gather_body,
        out_shape=(
            jax.ShapeDtypeStruct((2, T, D), jnp.float32),
            jax.ShapeDtypeStruct((2, T, 1), jnp.int32),
        ),
        in_specs=[
            pl.BlockSpec(memory_space=pltpu.VMEM),
            pl.BlockSpec(memory_space=pltpu.VMEM),
        ],
        out_specs=(
            pl.BlockSpec(memory_space=pltpu.VMEM),
            pl.BlockSpec(memory_space=pltpu.VMEM),
        ),
        scratch_shapes=[
            pltpu.SemaphoreType.DMA((2,)),
            pltpu.SemaphoreType.DMA((2,)),
        ],
        compiler_params=pltpu.CompilerParams(collective_id=0),
    )(x, assign2)

    xall2 = xall.reshape(2 * T, D)
    aall2 = aall.reshape(2 * T, 1)

    FT = 512
    NFT = F // FT

    def moe_body(xall_ref, aall_ref, w1_ref, w2_ref, out_ref, xm_ref):
        e = pl.program_id(0)
        ft = pl.program_id(1)
        my_y = lax.axis_index("y")

        @pl.when(ft == 0)
        def _():
            e_glob = my_y * E + e
            mask = aall_ref[...] == e_glob
            xm_ref[...] = jnp.where(mask, xall_ref[...], 0.0)

        @pl.when((e == 0) & (ft == 0))
        def _():
            out_ref[...] = jnp.zeros_like(out_ref)

        h = jnp.dot(xm_ref[...], w1_ref[0], preferred_element_type=jnp.float32)
        h = jnp.maximum(h, 0.0)
        out_ref[...] = out_ref[...] + jnp.dot(
            h, w2_ref[0], preferred_element_type=jnp.float32
        )

    contrib = pl.pallas_call(
        moe_body,
        grid=(E, NFT),
        out_shape=jax.ShapeDtypeStruct((2 * T, D), jnp.float32),
        in_specs=[
            pl.BlockSpec((2 * T, D), lambda e, ft: (0, 0)),
            pl.BlockSpec((2 * T, 1), lambda e, ft: (0, 0)),
            pl.BlockSpec((1, D, FT), lambda e, ft: (e, 0, ft)),
            pl.BlockSpec((1, FT, D), lambda e, ft: (e, ft, 0)),
        ],
        out_specs=pl.BlockSpec((2 * T, D), lambda e, ft: (0, 0)),
        scratch_shapes=[pltpu.VMEM((2 * T, D), jnp.float32)],
        compiler_params=pltpu.CompilerParams(
            dimension_semantics=("arbitrary", "arbitrary")
        ),
    )(xall2, aall2, W1, W2)

    contribs = contrib.reshape(2, T, D)

    def reduce_body(c_ref, out_ref, recv_ref, send_sem, recv_sem):
        _, peer = peer_of()
        peer_barrier(peer)

        rdma = pltpu.make_async_remote_copy(
            src_ref=c_ref.at[1],
            dst_ref=recv_ref,
            send_sem=send_sem,
            recv_sem=recv_sem,
            device_id=peer,
            device_id_type=pl.DeviceIdType.MESH,
        )
        rdma.start()
        rdma.wait()
        out_ref[...] = c_ref[0] + recv_ref[...]

    out = pl.pallas_call(
        reduce_body,
        out_shape=jax.ShapeDtypeStruct((T, D), jnp.float32),
        in_specs=[pl.BlockSpec(memory_space=pltpu.VMEM)],
        out_specs=pl.BlockSpec(memory_space=pltpu.VMEM),
        scratch_shapes=[
            pltpu.VMEM((T, D), jnp.float32),
            pltpu.SemaphoreType.DMA,
            pltpu.SemaphoreType.DMA,
        ],
        compiler_params=pltpu.CompilerParams(collective_id=1),
    )(contribs)

    return out


# device time: 127045 ns/iter; 1.1798x vs baseline; 1.1798x over previous
import jax
import jax.numpy as jnp
from jax import lax
from jax.experimental import pallas as pl
from jax.experimental.pallas import tpu as pltpu

C = 640


def kernel(x, assign, W1, W2):
    T, D = x.shape
    E, _, F = W1.shape

    my_y = lax.axis_index("y")

    owner = assign // 2
    to_peer = owner != my_y
    send_idx = jnp.argsort(jnp.where(to_peer, 0, 1), stable=True)[:C]
    loc_idx = jnp.argsort(jnp.where(to_peer, 1, 0), stable=True)[:C]

    xs = x[send_idx].astype(jnp.bfloat16)
    as_s = assign[send_idx].reshape(C, 1)
    xl = x[loc_idx].astype(jnp.bfloat16)
    as_l = assign[loc_idx].reshape(C, 1)

    def peer_of():
        return (
            lax.axis_index("x"),
            1 - lax.axis_index("y"),
            lax.axis_index("z"),
        )

    def peer_barrier(peer):
        barrier_sem = pltpu.get_barrier_semaphore()
        pl.semaphore_signal(
            barrier_sem,
            inc=1,
            device_id=peer,
            device_id_type=pl.DeviceIdType.MESH,
        )
        pl.semaphore_wait(barrier_sem, 1)

    def gather_body(xs_ref, as_ref, xr_ref, ar_ref, send_sems, recv_sems):
        peer = peer_of()
        peer_barrier(peer)

        rdma_x = pltpu.make_async_remote_copy(
            src_ref=xs_ref,
            dst_ref=xr_ref,
            send_sem=send_sems.at[0],
            recv_sem=recv_sems.at[0],
            device_id=peer,
            device_id_type=pl.DeviceIdType.MESH,
        )
        rdma_a = pltpu.make_async_remote_copy(
            src_ref=as_ref,
            dst_ref=ar_ref,
            send_sem=send_sems.at[1],
            recv_sem=recv_sems.at[1],
            device_id=peer,
            device_id_type=pl.DeviceIdType.MESH,
        )
        rdma_x.start()
        rdma_a.start()
        rdma_x.wait()
        rdma_a.wait()

    xr, ar = pl.pallas_call(
        gather_body,
        out_shape=(
            jax.ShapeDtypeStruct((C, D), jnp.bfloat16),
            jax.ShapeDtypeStruct((C, 1), jnp.int32),
        ),
        in_specs=[
            pl.BlockSpec(memory_space=pltpu.VMEM),
            pl.BlockSpec(memory_space=pltpu.VMEM),
        ],
        out_specs=(
            pl.BlockSpec(memory_space=pltpu.VMEM),
            pl.BlockSpec(memory_space=pltpu.VMEM),
        ),
        scratch_shapes=[
            pltpu.SemaphoreType.DMA((2,)),
            pltpu.SemaphoreType.DMA((2,)),
        ],
        compiler_params=pltpu.CompilerParams(collective_id=0),
    )(xs, as_s)

    FT = 512
    NFT = F // FT
    W1b = W1.astype(jnp.bfloat16)
    W2b = W2.astype(jnp.bfloat16)

    def moe_body(
        xl_ref, al_ref, xr_ref, ar_ref, w1_ref, w2_ref,
        out_ref, xm_ref, acc_ref,
    ):
        e = pl.program_id(0)
        ft = pl.program_id(1)

        @pl.when(ft == 0)
        def _():
            e_glob = lax.axis_index("y") * E + e
            zero = jnp.bfloat16(0)
            xm_ref[0:C] = jnp.where(al_ref[...] == e_glob, xl_ref[...], zero)
            xm_ref[C : 2 * C] = jnp.where(
                ar_ref[...] == e_glob, xr_ref[...], zero
            )

        h = jnp.dot(xm_ref[...], w1_ref[0], preferred_element_type=jnp.float32)
        h = jnp.maximum(h, 0.0).astype(jnp.bfloat16)
        part = jnp.dot(h, w2_ref[0], preferred_element_type=jnp.float32)

        @pl.when((e == 0) & (ft == 0))
        def _():
            acc_ref[...] = part

        @pl.when((e != 0) | (ft != 0))
        def _():
            acc_ref[...] = acc_ref[...] + part

        @pl.when((e == E - 1) & (ft == NFT - 1))
        def _():
            out_ref[...] = acc_ref[...].astype(jnp.bfloat16)

    contrib = pl.pallas_call(
        moe_body,
        grid=(E, NFT),
        out_shape=jax.ShapeDtypeStruct((2 * C, D), jnp.bfloat16),
        in_specs=[
            pl.BlockSpec((C, D), lambda e, ft: (0, 0)),
            pl.BlockSpec((C, 1), lambda e, ft: (0, 0)),
            pl.BlockSpec((C, D), lambda e, ft: (0, 0)),
            pl.BlockSpec((C, 1), lambda e, ft: (0, 0)),
            pl.BlockSpec((1, D, FT), lambda e, ft: (e, 0, ft)),
            pl.BlockSpec((1, FT, D), lambda e, ft: (e, ft, 0)),
        ],
        out_specs=pl.BlockSpec((2 * C, D), lambda e, ft: (0, 0)),
        scratch_shapes=[
            pltpu.VMEM((2 * C, D), jnp.bfloat16),
            pltpu.VMEM((2 * C, D), jnp.float32),
        ],
        compiler_params=pltpu.CompilerParams(
            dimension_semantics=("arbitrary", "arbitrary")
        ),
    )(xl, as_l, xr, ar, W1b, W2b)

    contribs = contrib.reshape(2, C, D)

    def reduce_body(c_ref, recv_ref, send_sem, recv_sem):
        peer = peer_of()
        peer_barrier(peer)

        rdma = pltpu.make_async_remote_copy(
            src_ref=c_ref.at[1],
            dst_ref=recv_ref,
            send_sem=send_sem,
            recv_sem=recv_sem,
            device_id=peer,
            device_id_type=pl.DeviceIdType.MESH,
        )
        rdma.start()
        rdma.wait()

    recvd = pl.pallas_call(
        reduce_body,
        out_shape=jax.ShapeDtypeStruct((C, D), jnp.bfloat16),
        in_specs=[pl.BlockSpec(memory_space=pltpu.VMEM)],
        out_specs=pl.BlockSpec(memory_space=pltpu.VMEM),
        scratch_shapes=[
            pltpu.SemaphoreType.DMA,
            pltpu.SemaphoreType.DMA,
        ],
        compiler_params=pltpu.CompilerParams(collective_id=1),
    )(contribs)

    out = (
        jnp.zeros((T, D), jnp.float32)
        .at[loc_idx]
        .add(contribs[0].astype(jnp.float32))
        .at[send_idx]
        .add(recvd.astype(jnp.float32))
    )
    return out
